# Initial kernel scaffold; baseline (speedup 1.0000x reference)
import jax
import jax.numpy as jnp
from jax.experimental import pallas as pl


def kernel(inputs):
    raise NotImplementedError("write your pallas kernel here")


# speedup vs baseline: 13.5532x; 13.5532x over previous
import jax
import jax.numpy as jnp
from jax.experimental import pallas as pl
from jax.experimental.pallas import tpu as pltpu

_KMAX = 50
_KMIN = 50
_SIGN_MASK = 0x7FFFFFFF
_INT_MIN = -2147483648
_INT_MAX = 2147483647


def _select_body(x_ref, o_ref, keys_ref):
    rows = x_ref.shape[0]
    bits = jax.lax.bitcast_convert_type(x_ref[...], jnp.int32)
    keys_ref[...] = jnp.where(bits >= 0, bits, bits ^ _SIGN_MASK)

    def counts(t, u):
        k = keys_ref[...]
        ct = jnp.sum((k >= t).astype(jnp.int32), axis=1, keepdims=True)
        cu = jnp.sum((k <= u).astype(jnp.int32), axis=1, keepdims=True)
        return ct, cu

    zero = jnp.zeros((rows, 1), jnp.int32)
    ct0, cu0 = counts(zero, zero - 1)
    t0 = jnp.where(ct0 >= _KMAX, zero, zero + _INT_MIN)
    u0 = jnp.where(cu0 >= _KMIN, zero - 1, zero + _INT_MAX)

    def body(i, carry):
        t, u = carry
        p = (1073741824 >> i).astype(jnp.int32)
        tt = t + p
        uu = u - p
        ct, cu = counts(tt, uu)
        return (jnp.where(ct >= _KMAX, tt, t), jnp.where(cu >= _KMIN, uu, u))

    t, u = jax.lax.fori_loop(0, 31, body, (t0, u0))

    k = keys_ref[...]
    x = x_ref[...]
    gt = k > t
    lt = k < u
    cnt_gt = jnp.sum(gt.astype(jnp.int32), axis=1, keepdims=True)
    cnt_lt = jnp.sum(lt.astype(jnp.int32), axis=1, keepdims=True)
    s_gt = jnp.sum(jnp.where(gt, x, 0.0), axis=1, keepdims=True)
    s_lt = jnp.sum(jnp.where(lt, x, 0.0), axis=1, keepdims=True)
    tval = jax.lax.bitcast_convert_type(
        jnp.where(t >= 0, t, t ^ _SIGN_MASK), jnp.float32)
    uval = jax.lax.bitcast_convert_type(
        jnp.where(u >= 0, u, u ^ _SIGN_MASK), jnp.float32)
    top = s_gt + (_KMAX - cnt_gt).astype(jnp.float32) * tval
    bot = s_lt + (_KMIN - cnt_lt).astype(jnp.float32) * uval
    res = top / _KMAX + bot / _KMIN
    o_ref[0] = jnp.broadcast_to(res, (rows, 128))


def kernel(inputs):
    b, h, w, c = inputs.shape
    n = h * w
    rows = b * c
    rg = 8
    assert rows % rg == 0 and n % 128 == 0
    g = rows // rg
    x = jnp.transpose(inputs, (0, 3, 1, 2)).reshape(rows, n)
    out = pl.pallas_call(
        _select_body,
        grid=(g,),
        in_specs=[pl.BlockSpec((rg, n), lambda i: (i, 0))],
        out_specs=pl.BlockSpec((1, rg, 128), lambda i: (i, 0, 0)),
        out_shape=jax.ShapeDtypeStruct((g, rg, 128), jnp.float32),
        scratch_shapes=[pltpu.VMEM((rg, n), jnp.int32)],
    )(x)
    return out[:, :, 0].reshape(b, c)


# speedup vs baseline: 19.4965x; 1.4385x over previous
import jax
import jax.numpy as jnp
from jax.experimental import pallas as pl
from jax.experimental.pallas import tpu as pltpu

_KMAX = 50
_KMIN = 50
_SIGN_MASK = 0x7FFFFFFF
_INT_MIN = -2147483648
_INT_MAX = 2147483647


_NSPLIT = 8


def _select_body(x_ref, o_ref, keys_ref):
    rows, n = x_ref.shape
    bits = jax.lax.bitcast_convert_type(x_ref[...], jnp.int32)
    keys_ref[...] = jnp.where(bits >= 0, bits, bits ^ _SIGN_MASK)
    seg = n // _NSPLIT

    def counts(t, u):
        cts, cus = [], []
        for s in range(_NSPLIT):
            k = keys_ref[:, s * seg:(s + 1) * seg]
            cts.append(jnp.sum((k >= t).astype(jnp.int32), axis=1,
                               keepdims=True))
            cus.append(jnp.sum((k <= u).astype(jnp.int32), axis=1,
                               keepdims=True))
        return sum(cts), sum(cus)

    zero = jnp.zeros((rows, 1), jnp.int32)
    ct0, cu0 = counts(zero, zero - 1)
    t0 = jnp.where(ct0 >= _KMAX, zero, zero + _INT_MIN)
    u0 = jnp.where(cu0 >= _KMIN, zero - 1, zero + _INT_MAX)

    def body(i, carry):
        t, u = carry
        p = (1073741824 >> i).astype(jnp.int32)
        tt = t + p
        uu = u - p
        ct, cu = counts(tt, uu)
        return (jnp.where(ct >= _KMAX, tt, t), jnp.where(cu >= _KMIN, uu, u))

    t, u = jax.lax.fori_loop(0, 31, body, (t0, u0))

    k = keys_ref[...]
    x = x_ref[...]
    gt = k > t
    lt = k < u
    cnt_gt = jnp.sum(gt.astype(jnp.int32), axis=1, keepdims=True)
    cnt_lt = jnp.sum(lt.astype(jnp.int32), axis=1, keepdims=True)
    s_gt = jnp.sum(jnp.where(gt, x, 0.0), axis=1, keepdims=True)
    s_lt = jnp.sum(jnp.where(lt, x, 0.0), axis=1, keepdims=True)
    tval = jax.lax.bitcast_convert_type(
        jnp.where(t >= 0, t, t ^ _SIGN_MASK), jnp.float32)
    uval = jax.lax.bitcast_convert_type(
        jnp.where(u >= 0, u, u ^ _SIGN_MASK), jnp.float32)
    top = s_gt + (_KMAX - cnt_gt).astype(jnp.float32) * tval
    bot = s_lt + (_KMIN - cnt_lt).astype(jnp.float32) * uval
    res = top / _KMAX + bot / _KMIN
    o_ref[0] = jnp.broadcast_to(res, (rows, 128))


def kernel(inputs):
    b, h, w, c = inputs.shape
    n = h * w
    rows = b * c
    rg = 16
    assert rows % rg == 0 and n % 128 == 0
    g = rows // rg
    x = jnp.transpose(inputs, (0, 3, 1, 2)).reshape(rows, n)
    out = pl.pallas_call(
        _select_body,
        grid=(g,),
        in_specs=[pl.BlockSpec((rg, n), lambda i: (i, 0))],
        out_specs=pl.BlockSpec((1, rg, 128), lambda i: (i, 0, 0)),
        out_shape=jax.ShapeDtypeStruct((g, rg, 128), jnp.float32),
        scratch_shapes=[pltpu.VMEM((rg, n), jnp.int32)],
    )(x)
    return out[:, :, 0].reshape(b, c)
